# baseline (speedup 1.0000x reference)
import jax
import jax.numpy as jnp
from jax.experimental import pallas as pl

_BM = 400
_HALF = _BM // 2


def _gcn_body(x_ref, w_ref, b_ref, adj_a_ref, adj_b_ref, out_ref):
    w = w_ref[...]
    b = b_ref[...]
    ax_a = jnp.dot(adj_a_ref[...], x_ref[...], preferred_element_type=jnp.float32)
    out_ref[:_HALF, :] = (
        jnp.dot(ax_a, w, preferred_element_type=jnp.float32) + b
    )
    ax_b = jnp.dot(adj_b_ref[...], x_ref[...], preferred_element_type=jnp.float32)
    out_ref[_HALF:, :] = (
        jnp.dot(ax_b, w, preferred_element_type=jnp.float32) + b
    )


def kernel(adj, inputs, W, b):
    n, d_in = inputs.shape
    d_out = W.shape[1]
    grid = (pl.cdiv(n, _BM),)
    return pl.pallas_call(
        _gcn_body,
        grid=grid,
        in_specs=[
            pl.BlockSpec((n, d_in), lambda i: (0, 0)),
            pl.BlockSpec((d_in, d_out), lambda i: (0, 0)),
            pl.BlockSpec((1, d_out), lambda i: (0, 0)),
            pl.BlockSpec((_HALF, n), lambda i: (2 * i, 0)),
            pl.BlockSpec((_HALF, n), lambda i: (2 * i + 1, 0)),
        ],
        out_specs=pl.BlockSpec((_BM, d_out), lambda i: (i, 0)),
        out_shape=jax.ShapeDtypeStruct((n, d_out), jnp.float32),
    )(inputs, W, b.reshape(1, d_out), adj, adj)


# speedup vs baseline: 1.0986x; 1.0986x over previous
import jax
import jax.numpy as jnp
from jax.experimental import pallas as pl
from jax.experimental.pallas import tpu as pltpu

_BM = 400


def _gcn_body(x_ref, w_ref, b_ref, adj_ref, out_ref):
    ax = jnp.dot(adj_ref[...], x_ref[...], preferred_element_type=jnp.float32)
    out_ref[...] = (
        jnp.dot(ax, w_ref[...], preferred_element_type=jnp.float32) + b_ref[...]
    )


def kernel(adj, inputs, W, b):
    n, d_in = inputs.shape
    d_out = W.shape[1]
    grid = (pl.cdiv(n, _BM),)
    return pl.pallas_call(
        _gcn_body,
        grid=grid,
        in_specs=[
            pl.BlockSpec((n, d_in), lambda i: (0, 0)),
            pl.BlockSpec((d_in, d_out), lambda i: (0, 0)),
            pl.BlockSpec((1, d_out), lambda i: (0, 0)),
            pl.BlockSpec((_BM, n), lambda i: (i, 0)),
        ],
        out_specs=pl.BlockSpec((_BM, d_out), lambda i: (i, 0)),
        out_shape=jax.ShapeDtypeStruct((n, d_out), jnp.float32),
        compiler_params=pltpu.CompilerParams(
            dimension_semantics=("parallel",),
        ),
    )(inputs, W, b.reshape(1, d_out), adj)


# speedup vs baseline: 1.1052x; 1.0060x over previous
import jax
import jax.numpy as jnp
from jax.experimental import pallas as pl

_BM = 400


def _gcn_body(x_ref, w_ref, b_ref, adj_ref, out_ref):
    ax = jnp.dot(adj_ref[...], x_ref[...], preferred_element_type=jnp.float32)
    out_ref[...] = (
        jnp.dot(ax, w_ref[...], preferred_element_type=jnp.float32) + b_ref[...]
    )


def kernel(adj, inputs, W, b):
    n, d_in = inputs.shape
    d_out = W.shape[1]
    grid = (pl.cdiv(n, _BM),)
    return pl.pallas_call(
        _gcn_body,
        grid=grid,
        in_specs=[
            pl.BlockSpec((n, d_in), lambda i: (0, 0)),
            pl.BlockSpec((d_in, d_out), lambda i: (0, 0)),
            pl.BlockSpec((1, d_out), lambda i: (0, 0)),
            pl.BlockSpec((_BM, n), lambda i: (i, 0)),
        ],
        out_specs=pl.BlockSpec((_BM, d_out), lambda i: (i, 0)),
        out_shape=jax.ShapeDtypeStruct((n, d_out), jnp.float32),
    )(inputs, W, b.reshape(1, d_out), adj)
